# Initial kernel scaffold; baseline (speedup 1.0000x reference)
import jax
import jax.numpy as jnp
from jax.experimental import pallas as pl


def kernel(x, x_e, edge_index, ln_in_g, ln_in_b, W_in, b_in, ln0_g, ln0_b, W_hg, b_hg, W_lin, b_lin):
    raise NotImplementedError("write your pallas kernel here")


# speedup vs baseline: 5.5158x; 5.5158x over previous
import functools

import jax
import jax.numpy as jnp
from jax import lax
from jax.experimental import pallas as pl
from jax.experimental.pallas import tpu as pltpu
from jax.experimental.pallas import tpu_sc as plsc

N = 10000
H = 10000
E = 320000
D = 128

NC = 2
NS = 16
NW = NC * NS
CHUNK = 80
ROWS = E // CHUNK
RPW = ROWS // NW
SJ = 25
OUTER = RPW // SJ
HP = 10240
RPT = HP // NS
ZR = 128
CPT = HP // NS
F32 = jnp.float32
I32 = jnp.int32

_mesh = plsc.VectorSubcoreMesh(
    core_axis_name="c", subcore_axis_name="s", num_cores=NC, num_subcores=NS)


def _leaky(x):
  return jnp.where(x >= 0, x, 0.01 * x)


def _ln(x, g, b):
  mu = jnp.mean(x, axis=-1, keepdims=True)
  var = jnp.mean((x - mu) ** 2, axis=-1, keepdims=True)
  return (x - mu) * lax.rsqrt(var + 1e-5) * g + b


def _tc_front_body(x_ref, g1_ref, b1_ref, wi_ref, bi_ref, g2_ref, b2_ref,
                   wh_ref, o_ref):
  x = x_ref[...]
  h = _ln(x, g1_ref[...], b1_ref[...])
  h = lax.dot_general(h, wi_ref[...], (((1,), (1,)), ((), ())),
                      preferred_element_type=F32) + bi_ref[...]
  h = _leaky(h)
  h = _ln(h, g2_ref[...], b2_ref[...])
  o_ref[...] = lax.dot_general(h, wh_ref[...], (((1,), (1,)), ((), ())),
                               preferred_element_type=F32)


def _tc_front(x, g1, b1, wi, bi, g2, b2, wh):
  blk = 1000
  vec = pl.BlockSpec((D,), lambda i: (0,))
  mat = pl.BlockSpec((D, D), lambda i: (0, 0))
  return pl.pallas_call(
      _tc_front_body,
      grid=(N // blk,),
      in_specs=[pl.BlockSpec((blk, D), lambda i: (i, 0)),
                vec, vec, mat, vec, vec, vec, mat],
      out_specs=pl.BlockSpec((blk, D), lambda i: (i, 0)),
      out_shape=jax.ShapeDtypeStruct((N, D), F32),
  )(x, g1, b1, wi, bi, g2, b2, wh)


def _tc_scale_body(with_bias, p_ref, c_ref, b_ref, o_ref):
  p = p_ref[...]
  s = p[0] + p[1]
  c = c_ref[...]
  cnt = c[0] + c[1]
  inv = jnp.where(cnt > 0, 1.0 / cnt, 0.0)
  r = s * inv[:, None]
  if with_bias:
    r = _leaky(r + b_ref[...])
  o_ref[...] = r


def _tc_scale(part, cnt, bias, with_bias):
  blk = 1024
  return pl.pallas_call(
      functools.partial(_tc_scale_body, with_bias),
      grid=(HP // blk,),
      in_specs=[pl.BlockSpec((2, blk, D), lambda i: (0, i, 0)),
                pl.BlockSpec((2, blk), lambda i: (0, i)),
                pl.BlockSpec((D,), lambda i: (0,))],
      out_specs=pl.BlockSpec((blk, D), lambda i: (i, 0)),
      out_shape=jax.ShapeDtypeStruct((HP, D), F32),
  )(part, cnt, bias)


def _tc_final(aggT, w, b):
  blk = 1024
  def body(a_ref, w_ref, b_ref, o_ref):
    a = a_ref[...]
    a = jnp.where(a < jnp.inf, a, 0.0)
    o_ref[...] = lax.dot_general(a, w_ref[...], (((0,), (1,)), ((), ())),
                                 preferred_element_type=F32) + b_ref[...]
  return pl.pallas_call(
      body,
      grid=(HP // blk,),
      in_specs=[pl.BlockSpec((D, blk), lambda i: (0, i)),
                pl.BlockSpec((D, D), lambda i: (0, 0)),
                pl.BlockSpec((D,), lambda i: (0,))],
      out_specs=pl.BlockSpec((blk, D), lambda i: (i, 0)),
      out_shape=jax.ShapeDtypeStruct((HP, D), F32),
  )(aggT, w, b)


def _sc_gsa_body(do_counts, *refs):
  if do_counts:
    (table, gidx, sidx, zeros2d, zeros1d, ones1d,
     out, cntg, cnts,
     gi_v, si_v, rows_v, z2_v, z1_v, on_v, acc_sp, cg_sp, cs_sp) = refs
  else:
    (table, gidx, sidx, zeros2d,
     out,
     gi_v, si_v, rows_v, z2_v, acc_sp) = refs

  c = lax.axis_index("c")
  s = lax.axis_index("s")
  w = c * NS + s

  pltpu.sync_copy(zeros2d, z2_v)
  for r in range(RPT // ZR):
    pltpu.sync_copy(z2_v, acc_sp.at[pl.ds(s * RPT + r * ZR, ZR)])
  del w
  if do_counts:
    pltpu.sync_copy(zeros1d, z1_v)
    pltpu.sync_copy(ones1d, on_v)
    pltpu.sync_copy(z1_v, cg_sp.at[pl.ds(s * CPT, CPT)])
    pltpu.sync_copy(z1_v, cs_sp.at[pl.ds(s * CPT, CPT)])
  plsc.subcore_barrier()

  wq = c * NS + s
  for t in range(OUTER):
    pltpu.sync_copy(gidx.at[wq, t], gi_v)
    pltpu.sync_copy(sidx.at[wq, t], si_v)

    def step(j, _):
      pltpu.sync_copy(table.at[gi_v.at[j]], rows_v)
      pltpu.sync_copy(rows_v, acc_sp.at[si_v.at[j]], add=True)
      if do_counts:
        pltpu.sync_copy(on_v, cg_sp.at[gi_v.at[j]], add=True)
        pltpu.sync_copy(on_v, cs_sp.at[si_v.at[j]], add=True)
      return 0

    lax.fori_loop(0, SJ, step, 0)

  plsc.subcore_barrier()

  for r in range(RPT // ZR):
    off = s * RPT + r * ZR
    pltpu.sync_copy(acc_sp.at[pl.ds(off, ZR)], z2_v)
    pltpu.sync_copy(z2_v, out.at[c, pl.ds(off, ZR)])
  if do_counts:
    pltpu.sync_copy(cg_sp.at[pl.ds(s * CPT, CPT)], z1_v)
    pltpu.sync_copy(z1_v, cntg.at[c, pl.ds(s * CPT, CPT)])
    pltpu.sync_copy(cs_sp.at[pl.ds(s * CPT, CPT)], z1_v)
    pltpu.sync_copy(z1_v, cnts.at[c, pl.ds(s * CPT, CPT)])


def _sc_gather_scatter_add(table, gidx2d, sidx2d, zeros2d, zeros1d, ones1d,
                           do_counts):
  out_types = [jax.ShapeDtypeStruct((NC, HP, D), F32)]
  scratch = [
      pltpu.VMEM((SJ, CHUNK), I32),
      pltpu.VMEM((SJ, CHUNK), I32),
      pltpu.VMEM((CHUNK, D), F32),
      pltpu.VMEM((ZR, D), F32),
  ]
  if do_counts:
    out_types += [jax.ShapeDtypeStruct((NC, HP), F32),
                  jax.ShapeDtypeStruct((NC, HP), F32)]
    scratch += [pltpu.VMEM((CPT,), F32), pltpu.VMEM((CHUNK,), F32)]
  scratch += [pltpu.VMEM_SHARED((HP, D), F32)]
  if do_counts:
    scratch += [pltpu.VMEM_SHARED((HP,), F32), pltpu.VMEM_SHARED((HP,), F32)]

  fn = pl.kernel(
      functools.partial(_sc_gsa_body, do_counts),
      out_type=tuple(out_types) if do_counts else out_types[0],
      mesh=_mesh,
      scratch_types=scratch,
  )
  if do_counts:
    return fn(table, gidx2d, sidx2d, zeros2d, zeros1d, ones1d)
  return fn(table, gidx2d, sidx2d, zeros2d)


FPT = D // NW


def _sc_min_body(h2, nidx, hidx, inf1d,
                 outT,
                 ni_v, hi_v, rb_v, col_v, acc_v, scr_v):
  c = lax.axis_index("c")
  s = lax.axis_index("s")
  w = c * NS + s
  iota = lax.iota(I32, 16)
  base_col = w * FPT

  def stage(k, _):
    pltpu.sync_copy(h2.at[pl.ds(k * 80 * D, 80 * D)], rb_v)

    def grp(g, _):
      r = g * 16 + iota
      rbase = r * D + base_col
      for f in range(FPT):
        v = plsc.load_gather(rb_v, [rbase + f])
        plsc.store_scatter(col_v, [k * 80 + r + f * HP], v)
      return 0

    lax.fori_loop(0, 80 // 16, grp, 0)
    return 0

  lax.fori_loop(0, HP // 80, stage, 0)
  for f in range(FPT):
    pltpu.sync_copy(inf1d, acc_v.at[pl.ds(f * HP, HP)])

  def outer(t, _):
    base = t * 2000
    pltpu.sync_copy(nidx.at[pl.ds(base, 2000)], ni_v)
    pltpu.sync_copy(hidx.at[pl.ds(base, 2000)], hi_v)

    def group(gidx, _):
      off = gidx * 16
      nid = ni_v[pl.ds(off, 16)]
      hid = hi_v[pl.ds(off, 16)]
      plsc.store_scatter(scr_v, [hid], iota)
      rb = plsc.load_gather(scr_v, [hid])
      dup = jnp.any(rb != iota)
      vals = []
      for f in range(FPT):
        v = plsc.load_gather(col_v, [nid + f * HP])
        a = hid + f * HP
        cur = plsc.load_gather(acc_v, [a])
        plsc.store_scatter(acc_v, [a], jnp.minimum(cur, v))
        vals.append((a, v))

      @pl.when(dup)
      def _fix():
        for f in range(FPT):
          a, v = vals[f]
          def rnd(r, _):
            cur = plsc.load_gather(acc_v, [a])
            act = cur > v
            plsc.store_scatter(acc_v, [a], jnp.minimum(cur, v), mask=act)
            return 0
          lax.fori_loop(0, 16, rnd, 0)
      return 0

    lax.fori_loop(0, 2000 // 16, group, 0)
    return 0

  lax.fori_loop(0, E // 2000, outer, 0)

  pltpu.sync_copy(acc_v, outT.at[w])


def _sc_scatter_min(h2, nidx1d, hidx1d, inf1d):
  return pl.kernel(
      _sc_min_body,
      out_type=jax.ShapeDtypeStruct((NW, FPT * HP), F32),
      mesh=_mesh,
      compiler_params=pltpu.CompilerParams(needs_layout_passes=False),
      scratch_types=[
          pltpu.VMEM((2000,), I32),
          pltpu.VMEM((2000,), I32),
          pltpu.VMEM((80 * D,), F32),
          pltpu.VMEM((FPT * HP,), F32),
          pltpu.VMEM((FPT * HP,), F32),
          pltpu.VMEM((H,), I32),
      ],
  )(h2, nidx1d, hidx1d, inf1d)


def kernel(x, x_e, edge_index, ln_in_g, ln_in_b, W_in, b_in, ln0_g, ln0_b,
           W_hg, b_hg, W_lin, b_lin):
  del x_e
  nidx = edge_index[0]
  hidx = edge_index[1]
  nidx2d = nidx.reshape(NW, OUTER, SJ, CHUNK)
  hidx2d = hidx.reshape(NW, OUTER, SJ, CHUNK)

  zeros2d = jnp.zeros((ZR, D), F32)
  zeros1d = jnp.zeros((CPT,), F32)
  ones1d = jnp.ones((CHUNK,), F32)
  inf1d = jnp.full((HP,), jnp.inf, F32)

  hx = _tc_front(x, ln_in_g, ln_in_b, W_in, b_in, ln0_g, ln0_b, W_hg)

  sum_e, cnt_n, cnt_h = _sc_gather_scatter_add(
      hx, nidx2d, hidx2d, zeros2d, zeros1d, ones1d, do_counts=True)
  out_e = _tc_scale(sum_e, cnt_h, b_hg, with_bias=False)

  sum_v = _sc_gather_scatter_add(
      out_e, hidx2d, nidx2d, zeros2d, zeros1d, ones1d, do_counts=False)
  h2 = _tc_scale(sum_v, cnt_n, b_hg, with_bias=True)

  aggT = _sc_scatter_min(h2.reshape(HP * D), nidx, hidx, inf1d)
  return _tc_final(aggT.reshape(D, HP), W_lin, b_lin)[:N]


# speedup vs baseline: 5.7280x; 1.0385x over previous
import functools

import jax
import jax.numpy as jnp
from jax import lax
from jax.experimental import pallas as pl
from jax.experimental.pallas import tpu as pltpu
from jax.experimental.pallas import tpu_sc as plsc

N = 10000
H = 10000
E = 320000
D = 128

NC = 2
NS = 16
NW = NC * NS
CHUNK = 80
ROWS = E // CHUNK
RPW = ROWS // NW
SJ = 25
OUTER = RPW // SJ
HP = 10240
RPT = HP // NS
ZR = 128
CPT = HP // NS
F32 = jnp.float32
I32 = jnp.int32

_mesh = plsc.VectorSubcoreMesh(
    core_axis_name="c", subcore_axis_name="s", num_cores=NC, num_subcores=NS)


def _leaky(x):
  return jnp.where(x >= 0, x, 0.01 * x)


def _ln(x, g, b):
  mu = jnp.mean(x, axis=-1, keepdims=True)
  var = jnp.mean((x - mu) ** 2, axis=-1, keepdims=True)
  return (x - mu) * lax.rsqrt(var + 1e-5) * g + b


def _tc_front_body(x_ref, g1_ref, b1_ref, wi_ref, bi_ref, g2_ref, b2_ref,
                   wh_ref, o_ref):
  x = x_ref[...]
  h = _ln(x, g1_ref[...], b1_ref[...])
  h = lax.dot_general(h, wi_ref[...], (((1,), (1,)), ((), ())),
                      preferred_element_type=F32) + bi_ref[...]
  h = _leaky(h)
  h = _ln(h, g2_ref[...], b2_ref[...])
  o_ref[...] = lax.dot_general(h, wh_ref[...], (((1,), (1,)), ((), ())),
                               preferred_element_type=F32)


def _tc_front(x, g1, b1, wi, bi, g2, b2, wh):
  blk = 1000
  vec = pl.BlockSpec((D,), lambda i: (0,))
  mat = pl.BlockSpec((D, D), lambda i: (0, 0))
  return pl.pallas_call(
      _tc_front_body,
      grid=(N // blk,),
      in_specs=[pl.BlockSpec((blk, D), lambda i: (i, 0)),
                vec, vec, mat, vec, vec, vec, mat],
      out_specs=pl.BlockSpec((blk, D), lambda i: (i, 0)),
      out_shape=jax.ShapeDtypeStruct((N, D), F32),
  )(x, g1, b1, wi, bi, g2, b2, wh)


def _tc_scale_body(with_bias, p_ref, c_ref, b_ref, o_ref):
  p = p_ref[...]
  s = p[0] + p[1]
  c = c_ref[...]
  cnt = c[0] + c[1]
  inv = jnp.where(cnt > 0, 1.0 / cnt, 0.0)
  r = s * inv[:, None]
  if with_bias:
    r = _leaky(r + b_ref[...])
  o_ref[...] = r


def _tc_scale(part, cnt, bias, with_bias):
  blk = 1024
  return pl.pallas_call(
      functools.partial(_tc_scale_body, with_bias),
      grid=(HP // blk,),
      in_specs=[pl.BlockSpec((2, blk, D), lambda i: (0, i, 0)),
                pl.BlockSpec((2, blk), lambda i: (0, i)),
                pl.BlockSpec((D,), lambda i: (0,))],
      out_specs=pl.BlockSpec((blk, D), lambda i: (i, 0)),
      out_shape=jax.ShapeDtypeStruct((HP, D), F32),
  )(part, cnt, bias)


def _tc_final(aggT, w, b):
  blk = 1024
  def body(a_ref, w_ref, b_ref, o_ref):
    a = a_ref[...]
    a = jnp.where(a < jnp.inf, a, 0.0)
    o_ref[...] = lax.dot_general(a, w_ref[...], (((0,), (1,)), ((), ())),
                                 preferred_element_type=F32) + b_ref[...]
  return pl.pallas_call(
      body,
      grid=(HP // blk,),
      in_specs=[pl.BlockSpec((D, blk), lambda i: (0, i)),
                pl.BlockSpec((D, D), lambda i: (0, 0)),
                pl.BlockSpec((D,), lambda i: (0,))],
      out_specs=pl.BlockSpec((blk, D), lambda i: (i, 0)),
      out_shape=jax.ShapeDtypeStruct((HP, D), F32),
  )(aggT, w, b)


def _sc_gsa_body(do_counts, *refs):
  if do_counts:
    (table, gidx, sidx, zeros2d, zeros1d, ones1d,
     out, cntg, cnts,
     gi_v, si_v, rows_v, z2_v, z1_v, on_v, acc_sp, cg_sp, cs_sp) = refs
  else:
    (table, gidx, sidx, zeros2d,
     out,
     gi_v, si_v, rows_v, z2_v, acc_sp) = refs

  c = lax.axis_index("c")
  s = lax.axis_index("s")
  w = c * NS + s

  pltpu.sync_copy(zeros2d, z2_v)
  for r in range(RPT // ZR):
    pltpu.sync_copy(z2_v, acc_sp.at[pl.ds(s * RPT + r * ZR, ZR)])
  del w
  if do_counts:
    pltpu.sync_copy(zeros1d, z1_v)
    pltpu.sync_copy(ones1d, on_v)
    pltpu.sync_copy(z1_v, cg_sp.at[pl.ds(s * CPT, CPT)])
    pltpu.sync_copy(z1_v, cs_sp.at[pl.ds(s * CPT, CPT)])
  plsc.subcore_barrier()

  wq = c * NS + s
  for t in range(OUTER):
    pltpu.sync_copy(gidx.at[wq, t], gi_v)
    pltpu.sync_copy(sidx.at[wq, t], si_v)

    def step(j, _):
      pltpu.sync_copy(table.at[gi_v.at[j]], rows_v)
      pltpu.sync_copy(rows_v, acc_sp.at[si_v.at[j]], add=True)
      if do_counts:
        pltpu.sync_copy(on_v, cg_sp.at[gi_v.at[j]], add=True)
        pltpu.sync_copy(on_v, cs_sp.at[si_v.at[j]], add=True)
      return 0

    lax.fori_loop(0, SJ, step, 0)

  plsc.subcore_barrier()

  for r in range(RPT // ZR):
    off = s * RPT + r * ZR
    pltpu.sync_copy(acc_sp.at[pl.ds(off, ZR)], z2_v)
    pltpu.sync_copy(z2_v, out.at[c, pl.ds(off, ZR)])
  if do_counts:
    pltpu.sync_copy(cg_sp.at[pl.ds(s * CPT, CPT)], z1_v)
    pltpu.sync_copy(z1_v, cntg.at[c, pl.ds(s * CPT, CPT)])
    pltpu.sync_copy(cs_sp.at[pl.ds(s * CPT, CPT)], z1_v)
    pltpu.sync_copy(z1_v, cnts.at[c, pl.ds(s * CPT, CPT)])


def _sc_gather_scatter_add(table, gidx2d, sidx2d, zeros2d, zeros1d, ones1d,
                           do_counts):
  out_types = [jax.ShapeDtypeStruct((NC, HP, D), F32)]
  scratch = [
      pltpu.VMEM((SJ, CHUNK), I32),
      pltpu.VMEM((SJ, CHUNK), I32),
      pltpu.VMEM((CHUNK, D), F32),
      pltpu.VMEM((ZR, D), F32),
  ]
  if do_counts:
    out_types += [jax.ShapeDtypeStruct((NC, HP), F32),
                  jax.ShapeDtypeStruct((NC, HP), F32)]
    scratch += [pltpu.VMEM((CPT,), F32), pltpu.VMEM((CHUNK,), F32)]
  scratch += [pltpu.VMEM_SHARED((HP, D), F32)]
  if do_counts:
    scratch += [pltpu.VMEM_SHARED((HP,), F32), pltpu.VMEM_SHARED((HP,), F32)]

  fn = pl.kernel(
      functools.partial(_sc_gsa_body, do_counts),
      out_type=tuple(out_types) if do_counts else out_types[0],
      mesh=_mesh,
      scratch_types=scratch,
  )
  if do_counts:
    return fn(table, gidx2d, sidx2d, zeros2d, zeros1d, ones1d)
  return fn(table, gidx2d, sidx2d, zeros2d)


FPT = D // NW


def _sc_min_body(h2, nidx, hidx, inf1d,
                 outT,
                 ni_v, hi_v, rb_v, col_v, acc_v, scr_v):
  c = lax.axis_index("c")
  s = lax.axis_index("s")
  w = c * NS + s
  iota = lax.iota(I32, 16)
  base_col = w * FPT

  def stage(k, _):
    pltpu.sync_copy(h2.at[pl.ds(k * 80 * D, 80 * D)], rb_v)

    def grp(g, _):
      r = g * 16 + iota
      rbase = r * D + base_col
      for f in range(FPT):
        v = plsc.load_gather(rb_v, [rbase + f])
        plsc.store_scatter(col_v, [k * 80 + r + f * HP], v)
      return 0

    lax.fori_loop(0, 80 // 16, grp, 0)
    return 0

  lax.fori_loop(0, HP // 80, stage, 0)
  for f in range(FPT):
    pltpu.sync_copy(inf1d, acc_v.at[pl.ds(f * HP, HP)])

  def outer(t, _):
    base = t * 2000
    pltpu.sync_copy(nidx.at[pl.ds(base, 2000)], ni_v)
    pltpu.sync_copy(hidx.at[pl.ds(base, 2000)], hi_v)

    def group(gidx, _):
      off = gidx * 16
      nid = ni_v[pl.ds(off, 16)]
      hid = hi_v[pl.ds(off, 16)]
      plsc.store_scatter(scr_v, [hid], iota)
      rb = plsc.load_gather(scr_v, [hid])
      dup = jnp.any(rb != iota)
      vals = []
      for f in range(FPT):
        v = plsc.load_gather(col_v, [nid + f * HP])
        a = hid + f * HP
        cur = plsc.load_gather(acc_v, [a])
        plsc.store_scatter(acc_v, [a], jnp.minimum(cur, v))
        vals.append((a, v))

      @pl.when(dup)
      def _fix():
        for f in range(FPT):
          a, v = vals[f]
          def rnd(r, _):
            cur = plsc.load_gather(acc_v, [a])
            act = cur > v
            plsc.store_scatter(acc_v, [a], jnp.minimum(cur, v), mask=act)
            return 0
          lax.fori_loop(0, 16, rnd, 0)
      return 0

    lax.fori_loop(0, 2000 // 16, group, 0, unroll=5)
    return 0

  lax.fori_loop(0, E // 2000, outer, 0)

  pltpu.sync_copy(acc_v, outT.at[w])


def _sc_scatter_min(h2, nidx1d, hidx1d, inf1d):
  return pl.kernel(
      _sc_min_body,
      out_type=jax.ShapeDtypeStruct((NW, FPT * HP), F32),
      mesh=_mesh,
      compiler_params=pltpu.CompilerParams(needs_layout_passes=False),
      scratch_types=[
          pltpu.VMEM((2000,), I32),
          pltpu.VMEM((2000,), I32),
          pltpu.VMEM((80 * D,), F32),
          pltpu.VMEM((FPT * HP,), F32),
          pltpu.VMEM((FPT * HP,), F32),
          pltpu.VMEM((H,), I32),
      ],
  )(h2, nidx1d, hidx1d, inf1d)


def kernel(x, x_e, edge_index, ln_in_g, ln_in_b, W_in, b_in, ln0_g, ln0_b,
           W_hg, b_hg, W_lin, b_lin):
  del x_e
  nidx = edge_index[0]
  hidx = edge_index[1]
  nidx2d = nidx.reshape(NW, OUTER, SJ, CHUNK)
  hidx2d = hidx.reshape(NW, OUTER, SJ, CHUNK)

  zeros2d = jnp.zeros((ZR, D), F32)
  zeros1d = jnp.zeros((CPT,), F32)
  ones1d = jnp.ones((CHUNK,), F32)
  inf1d = jnp.full((HP,), jnp.inf, F32)

  hx = _tc_front(x, ln_in_g, ln_in_b, W_in, b_in, ln0_g, ln0_b, W_hg)

  sum_e, cnt_n, cnt_h = _sc_gather_scatter_add(
      hx, nidx2d, hidx2d, zeros2d, zeros1d, ones1d, do_counts=True)
  out_e = _tc_scale(sum_e, cnt_h, b_hg, with_bias=False)

  sum_v = _sc_gather_scatter_add(
      out_e, hidx2d, nidx2d, zeros2d, zeros1d, ones1d, do_counts=False)
  h2 = _tc_scale(sum_v, cnt_n, b_hg, with_bias=True)

  aggT = _sc_scatter_min(h2.reshape(HP * D), nidx, hidx, inf1d)
  return _tc_final(aggT.reshape(D, HP), W_lin, b_lin)[:N]
